# Initial kernel scaffold; baseline (speedup 1.0000x reference)
import jax
import jax.numpy as jnp
from jax.experimental import pallas as pl


def kernel(query_emb, embeddings, start, end):
    raise NotImplementedError("write your pallas kernel here")


# speedup vs baseline: 1.0869x; 1.0869x over previous
import functools

import jax
import jax.numpy as jnp
from jax.experimental import pallas as pl
from jax.experimental.pallas import tpu as pltpu

_TOPK = 8
_BLK = 4096
_IMAX = 2**31 - 1


def _fused_topk_kernel(se_ref, q_ref, e_ref, vals_ref, idxs_ref, topv, topi,
                       *, k_total, blk, nq):
    step = pl.program_id(0)
    nsteps = pl.num_programs(0)

    @pl.when(step == 0)
    def _init():
        topv[...] = jnp.full((nq, _TOPK), -jnp.inf, dtype=jnp.float32)
        topi[...] = jnp.full((nq, _TOPK), -1, dtype=jnp.int32)

    q = q_ref[...]
    l1 = jnp.maximum(jnp.sum(jnp.abs(q), axis=1, keepdims=True), 1e-12)
    qn = q / l1
    scores = jnp.dot(qn, e_ref[...], preferred_element_type=jnp.float32)

    col = step * blk + jax.lax.broadcasted_iota(jnp.int32, (nq, blk), 1)
    start = se_ref[0]
    end = se_ref[1]
    bad = ((col >= start) & (col < end)) | (col >= k_total)
    scores = jnp.where(bad, -jnp.inf, scores)

    bv, bi = [], []
    for _ in range(_TOPK):
        m = jnp.max(scores, axis=1, keepdims=True)
        eq = scores == m
        mi = jnp.min(jnp.where(eq, col, _IMAX), axis=1, keepdims=True)
        bv.append(m)
        bi.append(mi)
        scores = jnp.where(col == mi, -jnp.inf, scores)
    bv = jnp.concatenate(bv, axis=1)
    bi = jnp.concatenate(bi, axis=1)

    cv = jnp.concatenate([topv[...], bv], axis=1)
    ci = jnp.concatenate([topi[...], bi], axis=1)
    nv, ni = [], []
    for _ in range(_TOPK):
        m = jnp.max(cv, axis=1, keepdims=True)
        eq = cv == m
        mi = jnp.min(jnp.where(eq, ci, _IMAX), axis=1, keepdims=True)
        nv.append(m)
        ni.append(mi)
        cv = jnp.where(eq & (ci == mi), -jnp.inf, cv)
    topv[...] = jnp.concatenate(nv, axis=1)
    topi[...] = jnp.concatenate(ni, axis=1)

    @pl.when(step == nsteps - 1)
    def _emit():
        vals_ref[...] = topv[...]
        idxs_ref[...] = topi[...]


def kernel(query_emb, embeddings, start, end):
    nq, d = query_emb.shape
    k_total = embeddings.shape[1]
    nsteps = pl.cdiv(k_total, _BLK)
    se = jnp.stack([jnp.asarray(start, jnp.int32), jnp.asarray(end, jnp.int32)])
    grid_spec = pltpu.PrefetchScalarGridSpec(
        num_scalar_prefetch=1,
        grid=(nsteps,),
        in_specs=[
            pl.BlockSpec((nq, d), lambda i, se_ref: (0, 0)),
            pl.BlockSpec((d, _BLK), lambda i, se_ref: (0, i)),
        ],
        out_specs=[
            pl.BlockSpec((nq, _TOPK), lambda i, se_ref: (0, 0)),
            pl.BlockSpec((nq, _TOPK), lambda i, se_ref: (0, 0)),
        ],
        scratch_shapes=[
            pltpu.VMEM((nq, _TOPK), jnp.float32),
            pltpu.VMEM((nq, _TOPK), jnp.int32),
        ],
    )
    vals, idxs = pl.pallas_call(
        functools.partial(_fused_topk_kernel, k_total=k_total, blk=_BLK, nq=nq),
        grid_spec=grid_spec,
        out_shape=[
            jax.ShapeDtypeStruct((nq, _TOPK), jnp.float32),
            jax.ShapeDtypeStruct((nq, _TOPK), jnp.int32),
        ],
        compiler_params=pltpu.CompilerParams(dimension_semantics=("arbitrary",)),
    )(se, query_emb, embeddings)
    return vals, idxs


# speedup vs baseline: 1.8935x; 1.7421x over previous
import functools

import jax
import jax.numpy as jnp
from jax.experimental import pallas as pl
from jax.experimental.pallas import tpu as pltpu

_TOPK = 8
_BLK = 4096
_IMAX = 2**31 - 1


def _fused_topk_kernel(se_ref, q_ref, e_ref, vals_ref, idxs_ref,
                       topv, topi, sc, bm, *, k_total, blk, nq):
    step = pl.program_id(0)
    nsteps = pl.num_programs(0)

    @pl.when(step == 0)
    def _init():
        topv[...] = jnp.full((nq, _TOPK), -jnp.inf, dtype=jnp.float32)
        topi[...] = jnp.full((nq, _TOPK), -1, dtype=jnp.int32)

    q = q_ref[...]
    l1 = jnp.maximum(jnp.sum(jnp.abs(q), axis=1, keepdims=True), 1e-12)
    qn = q / l1
    scores = jnp.dot(qn, e_ref[...], preferred_element_type=jnp.float32)

    col = step * blk + jax.lax.broadcasted_iota(jnp.int32, (nq, blk), 1)

    bstart = step * blk
    start = se_ref[0]
    end = se_ref[1]
    needs_mask = ((bstart < end) & (start < bstart + blk)) | (bstart + blk > k_total)

    @pl.when(needs_mask)
    def _masked():
        bad = ((col >= start) & (col < end)) | (col >= k_total)
        s = jnp.where(bad, -jnp.inf, scores)
        sc[...] = s
        bm[...] = jnp.max(s, axis=1, keepdims=True)

    @pl.when(jnp.logical_not(needs_mask))
    def _unmasked():
        sc[...] = scores
        bm[...] = jnp.max(scores, axis=1, keepdims=True)

    j8 = jax.lax.broadcasted_iota(jnp.int32, (nq, _TOPK), 1)
    for _ in range(_TOPK):
        improves = jnp.any(bm[...] > topv[...][:, _TOPK - 1:])

        @pl.when(improves)
        def _extract():
            s = sc[...]
            m = bm[...]
            eq = s == m
            mi = jnp.min(jnp.where(eq, col, _IMAX), axis=1, keepdims=True)
            ns = jnp.where(col == mi, -jnp.inf, s)
            sc[...] = ns
            bm[...] = jnp.max(ns, axis=1, keepdims=True)
            tv = topv[...]
            ti = topi[...]
            pos = jnp.sum((tv >= m).astype(jnp.int32), axis=1, keepdims=True)
            tvs = jnp.concatenate([tv[:, :1], tv[:, :_TOPK - 1]], axis=1)
            tis = jnp.concatenate([ti[:, :1], ti[:, :_TOPK - 1]], axis=1)
            topv[...] = jnp.where(j8 < pos, tv, jnp.where(j8 == pos, m, tvs))
            topi[...] = jnp.where(j8 < pos, ti, jnp.where(j8 == pos, mi, tis))

    @pl.when(step == nsteps - 1)
    def _emit():
        vals_ref[...] = topv[...]
        idxs_ref[...] = topi[...]


def kernel(query_emb, embeddings, start, end):
    nq, d = query_emb.shape
    k_total = embeddings.shape[1]
    nsteps = pl.cdiv(k_total, _BLK)
    se = jnp.stack([jnp.asarray(start, jnp.int32), jnp.asarray(end, jnp.int32)])
    grid_spec = pltpu.PrefetchScalarGridSpec(
        num_scalar_prefetch=1,
        grid=(nsteps,),
        in_specs=[
            pl.BlockSpec((nq, d), lambda i, se_ref: (0, 0)),
            pl.BlockSpec((d, _BLK), lambda i, se_ref: (0, i)),
        ],
        out_specs=[
            pl.BlockSpec((nq, _TOPK), lambda i, se_ref: (0, 0)),
            pl.BlockSpec((nq, _TOPK), lambda i, se_ref: (0, 0)),
        ],
        scratch_shapes=[
            pltpu.VMEM((nq, _TOPK), jnp.float32),
            pltpu.VMEM((nq, _TOPK), jnp.int32),
            pltpu.VMEM((nq, _BLK), jnp.float32),
            pltpu.VMEM((nq, 1), jnp.float32),
        ],
    )
    vals, idxs = pl.pallas_call(
        functools.partial(_fused_topk_kernel, k_total=k_total, blk=_BLK, nq=nq),
        grid_spec=grid_spec,
        out_shape=[
            jax.ShapeDtypeStruct((nq, _TOPK), jnp.float32),
            jax.ShapeDtypeStruct((nq, _TOPK), jnp.int32),
        ],
        compiler_params=pltpu.CompilerParams(dimension_semantics=("arbitrary",)),
    )(se, query_emb, embeddings)
    return vals, idxs


# speedup vs baseline: 3.2721x; 1.7281x over previous
import functools

import jax
import jax.numpy as jnp
from jax.experimental import pallas as pl
from jax.experimental.pallas import tpu as pltpu

_TOPK = 8
_BLK = 4096
_IMAX = 2**31 - 1


def _fused_topk_kernel(se_ref, q_ref, e_ref, vals_ref, idxs_ref,
                       topv, topi, sc, bm, qn_s, *, k_total, blk, nq):
    step = pl.program_id(0)
    nsteps = pl.num_programs(0)

    @pl.when(step == 0)
    def _init():
        topv[...] = jnp.full((nq, _TOPK), -jnp.inf, dtype=jnp.float32)
        topi[...] = jnp.full((nq, _TOPK), -1, dtype=jnp.int32)
        q = q_ref[...]
        l1 = jnp.maximum(jnp.sum(jnp.abs(q), axis=1, keepdims=True), 1e-12)
        qn_s[...] = q / l1

    scores = jnp.dot(qn_s[...], e_ref[...], preferred_element_type=jnp.float32)

    col = step * blk + jax.lax.broadcasted_iota(jnp.int32, (nq, blk), 1)

    bstart = step * blk
    start = se_ref[0]
    end = se_ref[1]
    needs_mask = ((bstart < end) & (start < bstart + blk)) | (bstart + blk > k_total)

    @pl.when(needs_mask)
    def _masked():
        bad = ((col >= start) & (col < end)) | (col >= k_total)
        s = jnp.where(bad, -jnp.inf, scores)
        sc[...] = s
        bm[...] = jnp.max(s, axis=1, keepdims=True)

    @pl.when(jnp.logical_not(needs_mask))
    def _unmasked():
        sc[...] = scores
        bm[...] = jnp.max(scores, axis=1, keepdims=True)

    j8 = jax.lax.broadcasted_iota(jnp.int32, (nq, _TOPK), 1)

    def _extract(_):
        s = sc[...]
        m = bm[...]
        eq = s == m
        mi = jnp.min(jnp.where(eq, col, _IMAX), axis=1, keepdims=True)
        ns = jnp.where(col == mi, -jnp.inf, s)
        sc[...] = ns
        bm[...] = jnp.max(ns, axis=1, keepdims=True)
        tv = topv[...]
        ti = topi[...]
        pos = jnp.sum((tv >= m).astype(jnp.int32), axis=1, keepdims=True)
        tvs = jnp.concatenate([tv[:, :1], tv[:, :_TOPK - 1]], axis=1)
        tis = jnp.concatenate([ti[:, :1], ti[:, :_TOPK - 1]], axis=1)
        topv[...] = jnp.where(j8 < pos, tv, jnp.where(j8 == pos, m, tvs))
        topi[...] = jnp.where(j8 < pos, ti, jnp.where(j8 == pos, mi, tis))
        return jnp.any(bm[...] > topv[...][:, _TOPK - 1:])

    jax.lax.while_loop(lambda c: c, _extract,
                       jnp.any(bm[...] > topv[...][:, _TOPK - 1:]))

    @pl.when(step == nsteps - 1)
    def _emit():
        vals_ref[...] = topv[...]
        idxs_ref[...] = topi[...]


def kernel(query_emb, embeddings, start, end):
    nq, d = query_emb.shape
    k_total = embeddings.shape[1]
    nsteps = pl.cdiv(k_total, _BLK)
    se = jnp.stack([jnp.asarray(start, jnp.int32), jnp.asarray(end, jnp.int32)])
    grid_spec = pltpu.PrefetchScalarGridSpec(
        num_scalar_prefetch=1,
        grid=(nsteps,),
        in_specs=[
            pl.BlockSpec((nq, d), lambda i, se_ref: (0, 0)),
            pl.BlockSpec((d, _BLK), lambda i, se_ref: (0, i)),
        ],
        out_specs=[
            pl.BlockSpec((nq, _TOPK), lambda i, se_ref: (0, 0)),
            pl.BlockSpec((nq, _TOPK), lambda i, se_ref: (0, 0)),
        ],
        scratch_shapes=[
            pltpu.VMEM((nq, _TOPK), jnp.float32),
            pltpu.VMEM((nq, _TOPK), jnp.int32),
            pltpu.VMEM((nq, _BLK), jnp.float32),
            pltpu.VMEM((nq, 1), jnp.float32),
            pltpu.VMEM((nq, d), jnp.float32),
        ],
    )
    vals, idxs = pl.pallas_call(
        functools.partial(_fused_topk_kernel, k_total=k_total, blk=_BLK, nq=nq),
        grid_spec=grid_spec,
        out_shape=[
            jax.ShapeDtypeStruct((nq, _TOPK), jnp.float32),
            jax.ShapeDtypeStruct((nq, _TOPK), jnp.int32),
        ],
        compiler_params=pltpu.CompilerParams(dimension_semantics=("arbitrary",)),
    )(se, query_emb, embeddings)
    return vals, idxs
